# Initial kernel scaffold; baseline (speedup 1.0000x reference)
import jax
import jax.numpy as jnp
from jax.experimental import pallas as pl
from jax.experimental.pallas import tpu as pltpu

N = 32768
D = 128
H1 = 64
E = 8
EH = E * H1


def _fused_kernel(x_ref, spec_ref, w1_ref, b1_ref, w2_ref, b2_ref, out_ref):
    x = x_ref[...]
    h = jnp.dot(x, w1_ref[...], preferred_element_type=jnp.float32)
    h = h + b1_ref[...]
    g = jax.nn.gelu(h, approximate=False)
    y = jnp.dot(g, w2_ref[...], preferred_element_type=jnp.float32)
    y = y + b2_ref[...]
    spec = spec_ref[...]
    lane = jax.lax.broadcasted_iota(jnp.int32, y.shape, 1)
    sel = jnp.where(lane == spec, y, 0.0)
    out_ref[...] = jnp.sum(sel, axis=1, keepdims=True)


def kernel(oc_density, species, W1, b1, W2, b2):
    n = oc_density.shape[0]
    B = 4096
    w1f = jnp.transpose(W1, (1, 0, 2)).reshape(D, EH)
    b1f = b1.reshape(1, EH)
    row_e = jnp.repeat(jnp.arange(E, dtype=jnp.int32), H1)
    w2bd = jnp.where(row_e[:, None] == jnp.arange(E, dtype=jnp.int32)[None, :],
                     W2[:, :, 0].reshape(EH, 1), 0.0)
    b2f = b2.reshape(1, E)
    spec2d = species.astype(jnp.int32).reshape(n, 1)

    grid = (n // B,)
    out = pl.pallas_call(
        _fused_kernel,
        grid=grid,
        in_specs=[
            pl.BlockSpec((B, D), lambda i: (i, 0)),
            pl.BlockSpec((B, 1), lambda i: (i, 0)),
            pl.BlockSpec((D, EH), lambda i: (0, 0)),
            pl.BlockSpec((1, EH), lambda i: (0, 0)),
            pl.BlockSpec((EH, E), lambda i: (0, 0)),
            pl.BlockSpec((1, E), lambda i: (0, 0)),
        ],
        out_specs=pl.BlockSpec((B, 1), lambda i: (i, 0)),
        out_shape=jax.ShapeDtypeStruct((n, 1), jnp.float32),
        compiler_params=pltpu.CompilerParams(
            dimension_semantics=("arbitrary",),
        ),
    )(oc_density, spec2d, w1f, b1f, w2bd, b2f)
    return out


# speedup vs baseline: 3.8111x; 3.8111x over previous
import jax
import jax.numpy as jnp
from jax.experimental import pallas as pl
from jax.experimental.pallas import tpu as pltpu

N = 32768
D = 128
H1 = 64
E = 8
EH = E * H1


def _fused_kernel(x_ref, spec_ref, w1_ref, b1_ref, w2_ref, b2_ref, out_ref):
    x = x_ref[...]
    h = jnp.dot(x, w1_ref[...], preferred_element_type=jnp.float32)
    h = h + b1_ref[...]
    g = 0.5 * h * (1.0 + jax.lax.erf(h * 0.7071067811865476))
    y = jnp.dot(g, w2_ref[...], preferred_element_type=jnp.float32)
    y = y + b2_ref[...]
    spec = spec_ref[...]
    lane = jax.lax.broadcasted_iota(jnp.int32, y.shape, 1)
    sel = jnp.where(lane == spec, y, 0.0)
    out_ref[...] = jnp.sum(sel, axis=1, keepdims=True)


def kernel(oc_density, species, W1, b1, W2, b2):
    n = oc_density.shape[0]
    B = 4096
    w1f = jnp.transpose(W1, (1, 0, 2)).reshape(D, EH)
    b1f = b1.reshape(1, EH)
    row_e = jnp.repeat(jnp.arange(E, dtype=jnp.int32), H1)
    w2bd = jnp.where(row_e[:, None] == jnp.arange(E, dtype=jnp.int32)[None, :],
                     W2[:, :, 0].reshape(EH, 1), 0.0)
    b2f = b2.reshape(1, E)
    spec2d = species.astype(jnp.int32).reshape(n, 1)

    grid = (n // B,)
    out = pl.pallas_call(
        _fused_kernel,
        grid=grid,
        in_specs=[
            pl.BlockSpec((B, D), lambda i: (i, 0)),
            pl.BlockSpec((B, 1), lambda i: (i, 0)),
            pl.BlockSpec((D, EH), lambda i: (0, 0)),
            pl.BlockSpec((1, EH), lambda i: (0, 0)),
            pl.BlockSpec((EH, E), lambda i: (0, 0)),
            pl.BlockSpec((1, E), lambda i: (0, 0)),
        ],
        out_specs=pl.BlockSpec((B, 1), lambda i: (i, 0)),
        out_shape=jax.ShapeDtypeStruct((n, 1), jnp.float32),
        compiler_params=pltpu.CompilerParams(
            dimension_semantics=("arbitrary",),
        ),
    )(oc_density, spec2d, w1f, b1f, w2bd, b2f)
    return out
